# baseline (speedup 1.0000x reference)
import jax
import jax.numpy as jnp
from jax import lax
from jax.experimental import pallas as pl
from jax.experimental.pallas import tpu as pltpu
from jax.experimental.pallas import tpu_sc as plsc

NC = 2
NS = 16
NW = NC * NS

D_MODEL = 64
TOTAL_ROWS = 4096 * 200
IDX_W = 128
N_IDX_ROWS = TOTAL_ROWS // IDX_W
ROWS_PER_W = N_IDX_ROWS // NW
G = 8


def _body(idx_hbm, table_hbm, out_hbm, idx_v, rows_v, sem):
    wid = lax.axis_index("s") * NC + lax.axis_index("c")
    base = wid * ROWS_PER_W

    def step(i, carry):
        row = base + i * G
        pltpu.sync_copy(idx_hbm.at[pl.ds(row, G)], idx_v)
        copies = [
            pltpu.async_copy(table_hbm.at[idx_v.at[j]], rows_v.at[j], sem)
            for j in range(G)
        ]
        for c in copies:
            c.wait()
        pltpu.sync_copy(rows_v, out_hbm.at[pl.ds(row, G)])
        return carry

    lax.fori_loop(0, ROWS_PER_W // G, step, 0)


_mesh = plsc.VectorSubcoreMesh(
    core_axis_name="c", subcore_axis_name="s", num_cores=NC, num_subcores=NS
)

_embed = pl.kernel(
    _body,
    out_type=jax.ShapeDtypeStruct((N_IDX_ROWS, IDX_W, D_MODEL), jnp.float32),
    mesh=_mesh,
    scratch_types=[
        pltpu.VMEM((G, IDX_W), jnp.int32),
        pltpu.VMEM((G, IDX_W, D_MODEL), jnp.float32),
        pltpu.SemaphoreType.DMA,
    ],
    compiler_params=pltpu.CompilerParams(use_tc_tiling_on_sc=False),
)


def kernel(word_indices, vocabulary):
    idx = jnp.reshape(word_indices.astype(jnp.int32), (N_IDX_ROWS, IDX_W))
    out = _embed(idx, vocabulary)
    return jnp.reshape(out, (*word_indices.shape, D_MODEL))


# speedup vs baseline: 1.0116x; 1.0116x over previous
import jax
import jax.numpy as jnp
from jax import lax
from jax.experimental import pallas as pl
from jax.experimental.pallas import tpu as pltpu
from jax.experimental.pallas import tpu_sc as plsc

NC = 2
NS = 16
NW = NC * NS

D_MODEL = 64
TOTAL_ROWS = 4096 * 200
IDX_W = 128
N_IDX_ROWS = TOTAL_ROWS // IDX_W
ROWS_PER_W = N_IDX_ROWS // NW
G = 4
NBUF = 2
N_CHUNKS = ROWS_PER_W // G
assert N_CHUNKS % NBUF == 0


def _body(idx_hbm, table_hbm, out_hbm, idx_v, rows0, rows1, g0, g1, s0, s1):
    wid = lax.axis_index("s") * NC + lax.axis_index("c")
    base = wid * ROWS_PER_W
    rows = [rows0, rows1]
    gsem = [g0, g1]
    ssem = [s0, s1]

    pltpu.sync_copy(idx_hbm.at[pl.ds(base, ROWS_PER_W)], idx_v)

    def outer(t, carry):
        gathers = []
        for b in range(NBUF):
            chunk = t * NBUF + b
            row = base + chunk * G

            @pl.when(t > 0)
            def _drain_store(b=b, row=row):
                pltpu.make_async_copy(rows[b], out_hbm.at[pl.ds(row, G)],
                                      ssem[b]).wait()

            gathers.append([
                pltpu.async_copy(table_hbm.at[idx_v.at[chunk * G + j]],
                                 rows[b].at[j], gsem[b])
                for j in range(G)
            ])
        for b in range(NBUF):
            chunk = t * NBUF + b
            row = base + chunk * G
            for c in gathers[b]:
                c.wait()
            pltpu.async_copy(rows[b], out_hbm.at[pl.ds(row, G)], ssem[b])
        return carry

    lax.fori_loop(0, N_CHUNKS // NBUF, outer, 0)

    for b in range(NBUF):
        row = base + (N_CHUNKS - NBUF + b) * G
        pltpu.make_async_copy(rows[b], out_hbm.at[pl.ds(row, G)], ssem[b]).wait()


_mesh = plsc.VectorSubcoreMesh(
    core_axis_name="c", subcore_axis_name="s", num_cores=NC, num_subcores=NS
)

_embed = pl.kernel(
    _body,
    out_type=jax.ShapeDtypeStruct((N_IDX_ROWS, IDX_W, D_MODEL), jnp.float32),
    mesh=_mesh,
    scratch_types=[
        pltpu.VMEM((ROWS_PER_W, IDX_W), jnp.int32),
        pltpu.VMEM((G, IDX_W, D_MODEL), jnp.float32),
        pltpu.VMEM((G, IDX_W, D_MODEL), jnp.float32),
        pltpu.SemaphoreType.DMA,
        pltpu.SemaphoreType.DMA,
        pltpu.SemaphoreType.DMA,
        pltpu.SemaphoreType.DMA,
    ],
    compiler_params=pltpu.CompilerParams(use_tc_tiling_on_sc=False),
)


def kernel(word_indices, vocabulary):
    idx = jnp.reshape(word_indices.astype(jnp.int32), (N_IDX_ROWS, IDX_W))
    out = _embed(idx, vocabulary)
    return jnp.reshape(out, (*word_indices.shape, D_MODEL))


# speedup vs baseline: 1.0138x; 1.0021x over previous
import jax
import jax.numpy as jnp
from jax import lax
from jax.experimental import pallas as pl
from jax.experimental.pallas import tpu as pltpu
from jax.experimental.pallas import tpu_sc as plsc

NC = 2
NS = 16
NW = NC * NS

B = 4096
S = 200
D_MODEL = 64
ROWS_PER_W = B // NW
CB = 4
NBUF = 2
N_STEPS = ROWS_PER_W // CB
assert N_STEPS % NBUF == 0
SPLITS = [(0, 128), (128, S - 128)]


def _body(idx_hbm, table_hbm, out_hbm, i0, i1, r0, r1, g0, g1, s0, s1):
    wid = lax.axis_index("s") * NC + lax.axis_index("c")
    base = wid * ROWS_PER_W
    idx_v = [i0, i1]
    rows = [r0, r1]
    gsem = [g0, g1]
    ssem = [s0, s1]

    def outer(t, carry):
        gathers = []
        for b in range(NBUF):
            row = base + (t * NBUF + b) * CB

            @pl.when(t > 0)
            def _drain_store(b=b, row=row):
                pltpu.make_async_copy(rows[b], out_hbm.at[pl.ds(row, CB)],
                                      ssem[b]).wait()

            pltpu.sync_copy(idx_hbm.at[pl.ds(row, CB)], idx_v[b])
            gathers.append([
                pltpu.async_copy(table_hbm.at[idx_v[b].at[r, pl.ds(o, w)]],
                                 rows[b].at[r, pl.ds(o, w)], gsem[b])
                for r in range(CB)
                for (o, w) in SPLITS
            ])
        for b in range(NBUF):
            row = base + (t * NBUF + b) * CB
            for c in gathers[b]:
                c.wait()
            pltpu.async_copy(rows[b], out_hbm.at[pl.ds(row, CB)], ssem[b])
        return carry

    lax.fori_loop(0, N_STEPS // NBUF, outer, 0)

    for b in range(NBUF):
        row = base + (N_STEPS - NBUF + b) * CB
        pltpu.make_async_copy(rows[b], out_hbm.at[pl.ds(row, CB)],
                              ssem[b]).wait()


_mesh = plsc.VectorSubcoreMesh(
    core_axis_name="c", subcore_axis_name="s", num_cores=NC, num_subcores=NS
)

_embed = pl.kernel(
    _body,
    out_type=jax.ShapeDtypeStruct((B, S, D_MODEL), jnp.float32),
    mesh=_mesh,
    scratch_types=[
        pltpu.VMEM((CB, S), jnp.int32),
        pltpu.VMEM((CB, S), jnp.int32),
        pltpu.VMEM((CB, S, D_MODEL), jnp.float32),
        pltpu.VMEM((CB, S, D_MODEL), jnp.float32),
        pltpu.SemaphoreType.DMA,
        pltpu.SemaphoreType.DMA,
        pltpu.SemaphoreType.DMA,
        pltpu.SemaphoreType.DMA,
    ],
    compiler_params=pltpu.CompilerParams(use_tc_tiling_on_sc=False),
)


def kernel(word_indices, vocabulary):
    return _embed(word_indices.astype(jnp.int32), vocabulary)
